# baseline (speedup 1.0000x reference)
import jax
import jax.numpy as jnp
from jax.experimental import pallas as pl
from jax.experimental.pallas import tpu as pltpu

_EPS = 1e-5


def _rup(x, m):
    return (x + m - 1) // m * m


def _row_tile(m, target):
    n_tiles = max(1, -(-m // target))
    return _rup(-(-m // n_tiles), 8)


def _dual_head_kernel(x_ref, wx_ref, y_ref, wy_ref, ox_ref, oy_ref):
    ox_ref[...] = jnp.dot(x_ref[...], wx_ref[...],
                          preferred_element_type=jnp.float32)
    oy_ref[...] = jnp.dot(y_ref[...], wy_ref[...],
                          preferred_element_type=jnp.float32)


def _conv_affine_kernel(p_ref, s_ref, t_ref, w_ref, o_ref):
    a = jnp.maximum(p_ref[...] * s_ref[...] + t_ref[...], 0.0)
    o_ref[...] = jnp.dot(a.astype(jnp.bfloat16), w_ref[...],
                         preferred_element_type=jnp.float32)


def _conv_affine_bias_tanh_kernel(p_ref, s_ref, t_ref, w_ref, b_ref, o_ref):
    a = jnp.maximum(p_ref[...] * s_ref[...] + t_ref[...], 0.0)
    r = jnp.dot(a.astype(jnp.bfloat16), w_ref[...],
                preferred_element_type=jnp.float32) + b_ref[...]
    o_ref[...] = jnp.tanh(r)


def _bn_coeffs(h_raw, gamma, beta):
    c = h_raw.shape[-1]
    h2 = h_raw.reshape(-1, c)
    mean = jnp.mean(h2, axis=0)
    var = jnp.mean(jnp.square(h2), axis=0) - jnp.square(mean)
    scale = gamma * jax.lax.rsqrt(var + _EPS)
    shift = beta - mean * scale
    return scale, shift


def _patches(h_raw, padval):
    n, h, w, c = h_raw.shape
    hb, wb = h + 1, w + 1
    xp = jnp.broadcast_to(padval, (n, h + 2, w + 2, c))
    xp = xp.at[:, 1:h + 1, 1:w + 1, :].set(h_raw)
    return jnp.concatenate(
        [xp[:, dy:dy + hb, dx:dx + wb, :] for dy in (0, 1) for dx in (0, 1)],
        axis=-1,
    ).reshape(n * hb * wb, 4 * c)


def _w_eff(w_pt):
    c_in, c_out = w_pt.shape[0], w_pt.shape[1]
    w6 = w_pt.reshape(c_in, c_out, 2, 2, 2, 2)[:, :, ::-1, :, ::-1, :]
    return jnp.transpose(w6, (2, 4, 0, 3, 5, 1)).reshape(
        4 * c_in, 4 * c_out).astype(jnp.bfloat16)


def _unshuffle(out, n, h, w, c_out):
    hb, wb = h + 1, w + 1
    out = out[:n * hb * wb].reshape(n, hb, wb, 2, 2, c_out)
    out = jnp.transpose(out, (0, 1, 3, 2, 4, 5)).reshape(
        n, 2 * hb, 2 * wb, c_out)
    return out[:, 1:2 * h + 1, 1:2 * w + 1, :]


def _deconv_layer(h_raw, scale, shift, w_pt, tm_target, bias=None):
    n, h, w, c_in = h_raw.shape
    c_out = w_pt.shape[1]
    patch = _patches(h_raw, -shift / scale)
    wm = _w_eff(w_pt)
    s4 = jnp.tile(scale, 4).reshape(1, 4 * c_in)
    t4 = jnp.tile(shift, 4).reshape(1, 4 * c_in)

    m = patch.shape[0]
    tm = _row_tile(m, tm_target)
    mp = _rup(m, tm)
    if mp != m:
        patch = jnp.pad(patch, ((0, mp - m), (0, 0)))

    in_specs = [
        pl.BlockSpec((tm, 4 * c_in), lambda i: (i, 0)),
        pl.BlockSpec((1, 4 * c_in), lambda i: (0, 0)),
        pl.BlockSpec((1, 4 * c_in), lambda i: (0, 0)),
        pl.BlockSpec((4 * c_in, 4 * c_out), lambda i: (0, 0)),
    ]
    args = [patch, s4, t4, wm]
    if bias is not None:
        in_specs.append(pl.BlockSpec((1, 4 * c_out), lambda i: (0, 0)))
        args.append(jnp.tile(bias, 4).reshape(1, 4 * c_out))
        body = _conv_affine_bias_tanh_kernel
    else:
        body = _conv_affine_kernel

    out = pl.pallas_call(
        body,
        out_shape=jax.ShapeDtypeStruct((mp, 4 * c_out), jnp.float32),
        grid=(mp // tm,),
        in_specs=in_specs,
        out_specs=pl.BlockSpec((tm, 4 * c_out), lambda i: (i, 0)),
        compiler_params=pltpu.CompilerParams(
            dimension_semantics=("parallel",)),
    )(*args)
    return _unshuffle(out, n, h, w, c_out)


def kernel(dc1_x_w, bn1_x_g, bn1_x_b, dc1_y_w, bn1_y_g, bn1_y_b,
           dc2_w, bn2_g, bn2_b, dc3_w, bn3_g, bn3_b,
           dc4_w, bn4_g, bn4_b, dc5_w, dc5_b, x, y):
    n, nz = x.shape[0], x.shape[1]
    ncls = y.shape[1]
    k = dc1_x_w.shape[2]
    c1 = dc1_x_w.shape[1]

    x2 = x.reshape(n, nz)
    y2 = y.reshape(n, ncls)
    wx = jnp.transpose(dc1_x_w, (0, 2, 3, 1)).reshape(nz, k * k * c1)
    wy = jnp.transpose(dc1_y_w, (0, 2, 3, 1)).reshape(ncls, k * k * c1)
    hx, hy = pl.pallas_call(
        _dual_head_kernel,
        out_shape=[jax.ShapeDtypeStruct((n, k * k * c1), jnp.float32),
                   jax.ShapeDtypeStruct((n, k * k * c1), jnp.float32)],
    )(x2, wx, y2, wy)
    hx = hx.reshape(n, k, k, c1)
    hy = hy.reshape(n, k, k, c1)

    sx, tx = _bn_coeffs(hx, bn1_x_g, bn1_x_b)
    sy, ty = _bn_coeffs(hy, bn1_y_g, bn1_y_b)
    h1 = jnp.concatenate([hx, hy], axis=-1)
    s1 = jnp.concatenate([sx, sy])
    t1 = jnp.concatenate([tx, ty])

    h2 = _deconv_layer(h1, s1, t1, dc2_w, tm_target=256)
    s2, t2 = _bn_coeffs(h2, bn2_g, bn2_b)
    h3 = _deconv_layer(h2, s2, t2, dc3_w, tm_target=512)
    s3, t3 = _bn_coeffs(h3, bn3_g, bn3_b)
    h4 = _deconv_layer(h3, s3, t3, dc4_w, tm_target=1024)
    s4, t4 = _bn_coeffs(h4, bn4_g, bn4_b)
    out = _deconv_layer(h4, s4, t4, dc5_w, tm_target=2048, bias=dc5_b)
    return jnp.transpose(out, (0, 3, 1, 2))


# speedup vs baseline: 2.5726x; 2.5726x over previous
import jax
import jax.numpy as jnp
from jax.experimental import pallas as pl
from jax.experimental.pallas import tpu as pltpu

_EPS = 1e-5


def _bn_coeffs(h_valid, gamma, beta):
    c = h_valid.shape[-1]
    h2 = h_valid.reshape(-1, c)
    mean = jnp.mean(h2, axis=0)
    var = jnp.mean(jnp.square(h2), axis=0) - jnp.square(mean)
    scale = gamma * jax.lax.rsqrt(var + _EPS)
    shift = beta - mean * scale
    return scale, shift


def _w_phase(w_pt):
    c_in, c_out = w_pt.shape[0], w_pt.shape[1]
    w6 = w_pt.reshape(c_in, c_out, 2, 2, 2, 2)[:, :, ::-1, :, ::-1, :]
    return jnp.transpose(w6, (2, 4, 0, 3, 5, 1)).reshape(
        4, c_in, 4 * c_out).astype(jnp.bfloat16)


def _deconv_body(x_ref, s_ref, t_ref, w_ref, o_ref, a_ref, *,
                 u_dim, v_dim, h_in, w_in, bias_ref=None, tanh=False):
    rows, c = x_ref.shape
    p = u_dim * v_dim
    r = jax.lax.broadcasted_iota(jnp.int32, (rows, 1), 0)
    u = (r // v_dim) % u_dim
    v = r % v_dim
    interior = (u >= 1) & (u <= h_in) & (v >= 1) & (v <= w_in)
    a = jnp.maximum(x_ref[...] * s_ref[...] + t_ref[...], 0.0)
    a_ref[...] = jnp.where(interior, a, 0.0).astype(jnp.bfloat16)

    l = rows - (v_dim + 1)
    acc = jnp.dot(a_ref[0:l, :], w_ref[0],
                  preferred_element_type=jnp.float32)
    acc += jnp.dot(a_ref[1:1 + l, :], w_ref[1],
                   preferred_element_type=jnp.float32)
    acc += jnp.dot(a_ref[v_dim:v_dim + l, :], w_ref[2],
                   preferred_element_type=jnp.float32)
    acc += jnp.dot(a_ref[v_dim + 1:v_dim + 1 + l, :], w_ref[3],
                   preferred_element_type=jnp.float32)
    if bias_ref is not None:
        acc += bias_ref[...]
    if tanh:
        acc = jnp.tanh(acc)
    o_ref[0:l, :] = acc
    o_ref[l:rows, :] = jnp.zeros((v_dim + 1, o_ref.shape[1]), jnp.float32)


def _deconv_layer(xg, scale, shift, w_pt, nb, bias=None, tanh=False):
    n, u_dim, v_dim, c_in = xg.shape
    nb = min(nb, n)
    h_in, w_in = u_dim - 2, v_dim - 2
    c_out = w_pt.shape[1]
    wm = _w_phase(w_pt)
    s2 = scale.reshape(1, c_in)
    t2 = shift.reshape(1, c_in)
    p = u_dim * v_dim
    rows = nb * p
    x2 = xg.reshape(n * p, c_in)

    in_specs = [
        pl.BlockSpec((rows, c_in), lambda i: (i, 0)),
        pl.BlockSpec((1, c_in), lambda i: (0, 0)),
        pl.BlockSpec((1, c_in), lambda i: (0, 0)),
        pl.BlockSpec((4, c_in, 4 * c_out), lambda i: (0, 0, 0)),
    ]
    args = [x2, s2, t2, wm]
    kw = dict(u_dim=u_dim, v_dim=v_dim, h_in=h_in, w_in=w_in, tanh=tanh)
    if bias is not None:
        in_specs.append(pl.BlockSpec((1, 4 * c_out), lambda i: (0, 0)))
        args.append(jnp.tile(bias, 4).reshape(1, 4 * c_out))

        def body(x_ref, s_ref, t_ref, w_ref, b_ref, o_ref, a_ref):
            _deconv_body(x_ref, s_ref, t_ref, w_ref, o_ref, a_ref,
                         bias_ref=b_ref, **kw)
    else:
        def body(x_ref, s_ref, t_ref, w_ref, o_ref, a_ref):
            _deconv_body(x_ref, s_ref, t_ref, w_ref, o_ref, a_ref, **kw)

    out = pl.pallas_call(
        body,
        out_shape=jax.ShapeDtypeStruct((n * p, 4 * c_out), jnp.float32),
        grid=(n // nb,),
        in_specs=in_specs,
        out_specs=pl.BlockSpec((rows, 4 * c_out), lambda i: (i, 0)),
        scratch_shapes=[pltpu.VMEM((rows, c_in), jnp.bfloat16)],
        compiler_params=pltpu.CompilerParams(
            dimension_semantics=("parallel",)),
    )(*args)

    hb, wb = h_in + 1, w_in + 1
    o6 = out.reshape(n, u_dim, v_dim, 2, 2, c_out)[:, :hb, :wb]
    o6 = jnp.transpose(o6, (0, 1, 3, 2, 4, 5))
    return o6.reshape(n, 2 * hb, 2 * wb, c_out)


def _dual_head_kernel(x_ref, wx_ref, y_ref, wy_ref, ox_ref, oy_ref):
    ox_ref[...] = jnp.dot(x_ref[...], wx_ref[...],
                          preferred_element_type=jnp.float32)
    oy_ref[...] = jnp.dot(y_ref[...], wy_ref[...],
                          preferred_element_type=jnp.float32)


def kernel(dc1_x_w, bn1_x_g, bn1_x_b, dc1_y_w, bn1_y_g, bn1_y_b,
           dc2_w, bn2_g, bn2_b, dc3_w, bn3_g, bn3_b,
           dc4_w, bn4_g, bn4_b, dc5_w, dc5_b, x, y):
    n, nz = x.shape[0], x.shape[1]
    ncls = y.shape[1]
    k = dc1_x_w.shape[2]
    c1 = dc1_x_w.shape[1]

    x2 = x.reshape(n, nz)
    y2 = y.reshape(n, ncls)
    wx = jnp.transpose(dc1_x_w, (0, 2, 3, 1)).reshape(nz, k * k * c1)
    wy = jnp.transpose(dc1_y_w, (0, 2, 3, 1)).reshape(ncls, k * k * c1)
    hx, hy = pl.pallas_call(
        _dual_head_kernel,
        out_shape=[jax.ShapeDtypeStruct((n, k * k * c1), jnp.float32),
                   jax.ShapeDtypeStruct((n, k * k * c1), jnp.float32)],
    )(x2, wx, y2, wy)
    hx = hx.reshape(n, k, k, c1)
    hy = hy.reshape(n, k, k, c1)

    sx, tx = _bn_coeffs(hx, bn1_x_g, bn1_x_b)
    sy, ty = _bn_coeffs(hy, bn1_y_g, bn1_y_b)
    h1 = jnp.concatenate([hx, hy], axis=-1)
    s1 = jnp.concatenate([sx, sy])
    t1 = jnp.concatenate([tx, ty])
    x0 = jnp.pad(h1, ((0, 0), (1, 1), (1, 1), (0, 0)))

    g2 = _deconv_layer(x0, s1, t1, dc2_w, nb=16)
    s2, t2 = _bn_coeffs(g2[:, 1:-1, 1:-1], bn2_g, bn2_b)
    g3 = _deconv_layer(g2, s2, t2, dc3_w, nb=8)
    s3, t3 = _bn_coeffs(g3[:, 1:-1, 1:-1], bn3_g, bn3_b)
    g4 = _deconv_layer(g3, s3, t3, dc4_w, nb=4)
    s4, t4 = _bn_coeffs(g4[:, 1:-1, 1:-1], bn4_g, bn4_b)
    g5 = _deconv_layer(g4, s4, t4, dc5_w, nb=2,
                       bias=dc5_b, tanh=True)
    img = g5[:, 1:-1, 1:-1, :]
    return jnp.transpose(img, (0, 3, 1, 2))


# speedup vs baseline: 3.9064x; 1.5185x over previous
import jax
import jax.numpy as jnp
from jax.experimental import pallas as pl
from jax.experimental.pallas import tpu as pltpu

_EPS = 1e-5


def _w_phase(w_pt):
    c_in, c_out = w_pt.shape[0], w_pt.shape[1]
    w6 = w_pt.reshape(c_in, c_out, 2, 2, 2, 2)[:, :, ::-1, :, ::-1, :]
    return jnp.transpose(w6, (2, 4, 0, 3, 5, 1)).reshape(
        4, c_in, 4 * c_out).astype(jnp.bfloat16)


def _affine_slab_acc(x_ref, s_ref, t_ref, w_ref, a_ref, u_dim, v_dim,
                     h_in, w_in):
    rows = x_ref.shape[0]
    r = jax.lax.broadcasted_iota(jnp.int32, (rows, 1), 0)
    u = (r // v_dim) % u_dim
    v = r % v_dim
    interior = (u >= 1) & (u <= h_in) & (v >= 1) & (v <= w_in)
    a = jnp.maximum(x_ref[...] * s_ref[...] + t_ref[...], 0.0)
    a_ref[...] = jnp.where(interior, a, 0.0).astype(jnp.bfloat16)

    l = rows - (v_dim + 1)
    acc = jnp.dot(a_ref[0:l, :], w_ref[0],
                  preferred_element_type=jnp.float32)
    acc += jnp.dot(a_ref[1:1 + l, :], w_ref[1],
                   preferred_element_type=jnp.float32)
    acc += jnp.dot(a_ref[v_dim:v_dim + l, :], w_ref[2],
                   preferred_element_type=jnp.float32)
    acc += jnp.dot(a_ref[v_dim + 1:v_dim + 1 + l, :], w_ref[3],
                   preferred_element_type=jnp.float32)
    return acc, l


def _mid_layer_body(x_ref, s_ref, t_ref, w_ref, o_ref, st_ref, a_ref, *,
                    nb, u_dim, v_dim, h_in, w_in, c_out):
    acc, l = _affine_slab_acc(x_ref, s_ref, t_ref, w_ref, a_ref,
                              u_dim, v_dim, h_in, w_in)
    hb, wb = h_in + 1, w_in + 1
    rows = nb * u_dim * v_dim
    acc_full = jnp.concatenate(
        [acc, jnp.zeros((rows - l, 4 * c_out), jnp.float32)], axis=0)
    acc4 = acc_full.reshape(nb, u_dim, v_dim, 4 * c_out)

    r = jax.lax.broadcasted_iota(jnp.int32, (l, 1), 0)
    i = (r // v_dim) % u_dim
    j = r % v_dim
    for g, (py, px) in enumerate(((0, 0), (0, 1), (1, 0), (1, 1))):
        val = acc4[:, 0:hb, 0:wb, g * c_out:(g + 1) * c_out]
        o_ref[:, :, py, :, px, :] = val
        m = ((2 * i + py >= 1) & (2 * i + py <= 2 * h_in) &
             (2 * j + px >= 1) & (2 * j + px <= 2 * w_in))
        accg = acc[:, g * c_out:(g + 1) * c_out]
        mg = jnp.where(m, accg, 0.0)
        st_ref[g, :] = jnp.sum(mg, axis=0)
        st_ref[4 + g, :] = jnp.sum(mg * accg, axis=0)


def _last_layer_body(x_ref, s_ref, t_ref, w_ref, b_ref, o_ref, a_ref, *,
                     u_dim, v_dim, h_in, w_in):
    acc, l = _affine_slab_acc(x_ref, s_ref, t_ref, w_ref, a_ref,
                              u_dim, v_dim, h_in, w_in)
    rows = x_ref.shape[0]
    acc = jnp.tanh(acc + b_ref[...])
    o_ref[0:l, :] = acc
    o_ref[l:rows, :] = jnp.zeros((v_dim + 1, o_ref.shape[1]), jnp.float32)


def _mid_layer(xg, scale, shift, w_pt, nb):
    n, u_dim, v_dim, c_in = xg.shape
    nb = min(nb, n)
    h_in, w_in = u_dim - 2, v_dim - 2
    hb, wb = h_in + 1, w_in + 1
    c_out = w_pt.shape[1]
    p = u_dim * v_dim
    rows = nb * p
    steps = n // nb

    def body(x_ref, s_ref, t_ref, w_ref, o_ref, st_ref, a_ref):
        _mid_layer_body(x_ref, s_ref, t_ref, w_ref, o_ref, st_ref, a_ref,
                        nb=nb, u_dim=u_dim, v_dim=v_dim, h_in=h_in,
                        w_in=w_in, c_out=c_out)

    out, st = pl.pallas_call(
        body,
        out_shape=[
            jax.ShapeDtypeStruct((n, hb, 2, wb, 2, c_out), jnp.float32),
            jax.ShapeDtypeStruct((steps * 8, c_out), jnp.float32),
        ],
        grid=(steps,),
        in_specs=[
            pl.BlockSpec((rows, c_in), lambda i: (i, 0)),
            pl.BlockSpec((1, c_in), lambda i: (0, 0)),
            pl.BlockSpec((1, c_in), lambda i: (0, 0)),
            pl.BlockSpec((4, c_in, 4 * c_out), lambda i: (0, 0, 0)),
        ],
        out_specs=[
            pl.BlockSpec((nb, hb, 2, wb, 2, c_out),
                         lambda i: (i, 0, 0, 0, 0, 0)),
            pl.BlockSpec((8, c_out), lambda i: (i, 0)),
        ],
        scratch_shapes=[pltpu.VMEM((rows, c_in), jnp.bfloat16)],
        compiler_params=pltpu.CompilerParams(
            dimension_semantics=("parallel",)),
    )(xg.reshape(n * p, c_in), scale.reshape(1, c_in),
      shift.reshape(1, c_in), _w_phase(w_pt))

    g_next = out.reshape(n, 2 * hb, 2 * wb, c_out)
    stp = st.reshape(steps, 8, c_out).sum(axis=0)
    count = n * (2 * h_in) * (2 * w_in)
    mean = (stp[0] + stp[1] + stp[2] + stp[3]) / count
    meansq = (stp[4] + stp[5] + stp[6] + stp[7]) / count
    return g_next, mean, meansq - jnp.square(mean)


def _last_layer(xg, scale, shift, w_pt, bias, nb):
    n, u_dim, v_dim, c_in = xg.shape
    nb = min(nb, n)
    h_in, w_in = u_dim - 2, v_dim - 2
    c_out = w_pt.shape[1]
    p = u_dim * v_dim
    rows = nb * p

    def body(x_ref, s_ref, t_ref, w_ref, b_ref, o_ref, a_ref):
        _last_layer_body(x_ref, s_ref, t_ref, w_ref, b_ref, o_ref, a_ref,
                         u_dim=u_dim, v_dim=v_dim, h_in=h_in, w_in=w_in)

    out = pl.pallas_call(
        body,
        out_shape=jax.ShapeDtypeStruct((n * p, 4 * c_out), jnp.float32),
        grid=(n // nb,),
        in_specs=[
            pl.BlockSpec((rows, c_in), lambda i: (i, 0)),
            pl.BlockSpec((1, c_in), lambda i: (0, 0)),
            pl.BlockSpec((1, c_in), lambda i: (0, 0)),
            pl.BlockSpec((4, c_in, 4 * c_out), lambda i: (0, 0, 0)),
            pl.BlockSpec((1, 4 * c_out), lambda i: (0, 0)),
        ],
        out_specs=pl.BlockSpec((rows, 4 * c_out), lambda i: (i, 0)),
        scratch_shapes=[pltpu.VMEM((rows, c_in), jnp.bfloat16)],
        compiler_params=pltpu.CompilerParams(
            dimension_semantics=("parallel",)),
    )(xg.reshape(n * p, c_in), scale.reshape(1, c_in),
      shift.reshape(1, c_in), _w_phase(w_pt),
      jnp.tile(bias, 4).reshape(1, 4 * c_out))

    hb, wb = h_in + 1, w_in + 1
    o6 = out.reshape(n, u_dim, v_dim, 2, 2, c_out)[:, :hb, :wb]
    o6 = jnp.transpose(o6, (0, 1, 3, 2, 4, 5)).reshape(
        n, 2 * hb, 2 * wb, c_out)
    return o6[:, 1:2 * h_in + 1, 1:2 * w_in + 1, :]


def _dual_head_kernel(x_ref, wx_ref, y_ref, wy_ref, ox_ref, oy_ref):
    ox_ref[...] = jnp.dot(x_ref[...], wx_ref[...],
                          preferred_element_type=jnp.float32)
    oy_ref[...] = jnp.dot(y_ref[...], wy_ref[...],
                          preferred_element_type=jnp.float32)


def _bn_coeffs_from(mean, var, gamma, beta):
    scale = gamma * jax.lax.rsqrt(var + _EPS)
    return scale, beta - mean * scale


def kernel(dc1_x_w, bn1_x_g, bn1_x_b, dc1_y_w, bn1_y_g, bn1_y_b,
           dc2_w, bn2_g, bn2_b, dc3_w, bn3_g, bn3_b,
           dc4_w, bn4_g, bn4_b, dc5_w, dc5_b, x, y):
    n, nz = x.shape[0], x.shape[1]
    ncls = y.shape[1]
    k = dc1_x_w.shape[2]
    c1 = dc1_x_w.shape[1]

    x2 = x.reshape(n, nz)
    y2 = y.reshape(n, ncls)
    wx = jnp.transpose(dc1_x_w, (0, 2, 3, 1)).reshape(nz, k * k * c1)
    wy = jnp.transpose(dc1_y_w, (0, 2, 3, 1)).reshape(ncls, k * k * c1)
    hx, hy = pl.pallas_call(
        _dual_head_kernel,
        out_shape=[jax.ShapeDtypeStruct((n, k * k * c1), jnp.float32),
                   jax.ShapeDtypeStruct((n, k * k * c1), jnp.float32)],
    )(x2, wx, y2, wy)

    def head_stats(h, gamma, beta):
        m = jnp.mean(h.reshape(-1, c1), axis=0)
        v = jnp.mean(jnp.square(h.reshape(-1, c1)), axis=0) - jnp.square(m)
        return _bn_coeffs_from(m, v, gamma, beta)

    sx, tx = head_stats(hx, bn1_x_g, bn1_x_b)
    sy, ty = head_stats(hy, bn1_y_g, bn1_y_b)
    h1 = jnp.concatenate([hx.reshape(n, k, k, c1),
                          hy.reshape(n, k, k, c1)], axis=-1)
    s1 = jnp.concatenate([sx, sy])
    t1 = jnp.concatenate([tx, ty])
    x0 = jnp.pad(h1, ((0, 0), (1, 1), (1, 1), (0, 0)))

    g2, m2, v2 = _mid_layer(x0, s1, t1, dc2_w, nb=16)
    s2, t2 = _bn_coeffs_from(m2, v2, bn2_g, bn2_b)
    g3, m3, v3 = _mid_layer(g2, s2, t2, dc3_w, nb=8)
    s3, t3 = _bn_coeffs_from(m3, v3, bn3_g, bn3_b)
    g4, m4, v4 = _mid_layer(g3, s3, t3, dc4_w, nb=4)
    s4, t4 = _bn_coeffs_from(m4, v4, bn4_g, bn4_b)
    img = _last_layer(g4, s4, t4, dc5_w, dc5_b, nb=2)
    return jnp.transpose(img, (0, 3, 1, 2))
